# baseline (speedup 1.0000x reference)
import functools

import jax
import jax.numpy as jnp
from jax import lax
from jax.experimental import pallas as pl
from jax.experimental.pallas import tpu as pltpu
from jax.experimental.pallas import tpu_sc as plsc

_THRESHOLD = 0.1
_ONE_BITS = 0x3F800001


def _sigmoid(z):
    return 1.0 / (1.0 + jnp.exp(-z))


def _score_body(score_ref, ego_ref, cav_ref, out_ref):
    ego = ego_ref[0, 0]
    cav = cav_ref[0, 0]
    del score_ref
    r = jnp.max(_sigmoid(cav - ego), axis=0, keepdims=True)
    a = jnp.max(_sigmoid(cav + ego), axis=0, keepdims=True)
    out_ref[0, 0] = lax.bitcast_convert_type(
        jnp.concatenate([r, a], axis=0), jnp.int32)


def _scores(psm4, B, L, P, HW):
    ego_spec = pl.BlockSpec((1, 1, P, HW), lambda b, ip: (b, 0, 0, 0))
    cav_spec = pl.BlockSpec((1, 1, P, HW), lambda b, ip: (b, ip + 1, 0, 0))
    out_spec = pl.BlockSpec((1, 1, 2, HW), lambda b, ip: (b, ip, 0, 0))
    return pl.pallas_call(
        functools.partial(_score_body, None),
        grid=(B, L - 1),
        in_specs=[ego_spec, cav_spec],
        out_specs=out_spec,
        out_shape=jax.ShapeDtypeStruct((B, L - 1, 2, HW), jnp.int32),
        compiler_params=pltpu.CompilerParams(
            dimension_semantics=("arbitrary", "arbitrary")),
    )(psm4, psm4)


def _sc_thresholds(scores2, NMAPS, HW, K):
    info = plsc.get_sparse_core_info()
    NC = info.num_cores
    mesh = plsc.VectorSubcoreMesh(core_axis_name="c", subcore_axis_name="s")
    n_chunk = HW // 16
    n_outer = n_chunk // 16

    def body(scores_hbm, thr_hbm, buf, tv):
        wid = lax.axis_index("s") * NC + lax.axis_index("c")

        @pl.when(wid < NMAPS)
        def _():
            pltpu.sync_copy(scores_hbm.at[wid], buf)
            one = jnp.ones((16,), jnp.int32)
            zero = jnp.zeros((16,), jnp.int32)

            del one

            def count_ge(mid):
                def inner(j, acc):
                    base = j * 256
                    for t in range(16):
                        v = buf[pl.ds(base + t * 16, 16)]
                        acc = acc + plsc.all_reduce_population_count(v >= mid)
                    return acc

                return lax.fori_loop(0, n_outer, inner, zero)

            def step(_, lohi):
                lo, hi = lohi
                mid = lo + lax.shift_right_logical(hi - lo, 1)
                pred = count_ge(mid) >= K
                return (jnp.where(pred, mid, lo), jnp.where(pred, hi, mid))

            lo, _ = lax.fori_loop(
                0, 31, step, (zero, jnp.full((16,), _ONE_BITS, jnp.int32)))
            tv[...] = lo
            pltpu.sync_copy(tv, thr_hbm.at[wid])

    run = pl.kernel(
        body,
        out_type=jax.ShapeDtypeStruct((NMAPS, 16), jnp.int32),
        mesh=mesh,
        scratch_types=[pltpu.VMEM((HW,), jnp.int32),
                       pltpu.VMEM((16,), jnp.int32)],
        compiler_params=pltpu.CompilerParams(needs_layout_passes=False),
    )
    return run(scores2)


def _copy_body(x_ref, _s, of_ref, or_ref, oa_ref):
    xb = x_ref[0, 0]
    of_ref[0, 0] = xb
    or_ref[0, 0] = xb
    oa_ref[0, 0] = xb


def _gate_body(L, HW, CB, keep_ref, thr_ref, score_ref, x_ref, _a0, _a1, _a2,
               of_ref, or_ref, oa_ref, mm, mr, ma):
    b = pl.program_id(0)
    ip = pl.program_id(1)
    cb = pl.program_id(2)

    @pl.when(cb == 0)
    def _build_masks():
        bits = score_ref[0, 0]
        t = (b * (L - 1) + ip) * 2
        kf = jnp.where(keep_ref[b * L + ip + 1] != 0, jnp.float32(1.0),
                       jnp.float32(0.0))
        fr = (bits[0:1] >= thr_ref[t]).astype(jnp.float32) * kf
        fa = (bits[1:2] >= thr_ref[t + 1]).astype(jnp.float32) * kf
        mr[:] = jnp.broadcast_to(fr, (8, HW))
        ma[:] = jnp.broadcast_to(fa, (8, HW))
        mm[:] = jnp.broadcast_to(jnp.maximum(fr, fa), (8, HW))

    mmv = mm[:]
    mrv = mr[:]
    mav = ma[:]
    for j in range(CB // 8):
        sl = pl.ds(j * 8, 8)
        xs = x_ref[0, 0, sl]
        of_ref[0, 0, sl] = xs * mmv
        or_ref[0, 0, sl] = xs * mrv
        oa_ref[0, 0, sl] = xs * mav


def kernel(x, psm, mask, flag):
    B, L, C, H, W = x.shape
    P = psm.shape[2]
    HW = H * W
    K = max(1, int(HW * _THRESHOLD))
    CB = 64 if C % 64 == 0 else C
    NCB = C // CB
    NMAPS = B * (L - 1) * 2

    x4 = x.reshape(B, L, C, HW)
    psm4 = psm.reshape(B, L, P, HW)
    keep = ((mask * jnp.asarray(flag, mask.dtype)) != 0).astype(
        jnp.int32).reshape(-1)

    scores = _scores(psm4, B, L, P, HW)
    thr = _sc_thresholds(scores.reshape(NMAPS, HW), NMAPS, HW, K)
    thr_flat = thr[:, 0]

    out_shapes = [jax.ShapeDtypeStruct((B, L, C, HW), jnp.float32)] * 3

    c_x_spec = pl.BlockSpec((1, 1, CB, HW), lambda b, cb: (b, 0, cb, 0))
    outs0 = pl.pallas_call(
        _copy_body,
        grid=(B, NCB),
        in_specs=[c_x_spec, pl.BlockSpec(memory_space=pl.ANY)],
        out_specs=[c_x_spec, c_x_spec, c_x_spec],
        out_shape=out_shapes,
        compiler_params=pltpu.CompilerParams(
            dimension_semantics=("arbitrary", "arbitrary")),
    )(x4, scores)

    score_spec = pl.BlockSpec((1, 1, 2, HW), lambda b, ip, cb, *_: (b, ip, 0, 0))
    x_spec = pl.BlockSpec((1, 1, CB, HW), lambda b, ip, cb, *_: (b, ip + 1, cb, 0))
    any_spec = pl.BlockSpec(memory_space=pl.ANY)

    grid_spec = pltpu.PrefetchScalarGridSpec(
        num_scalar_prefetch=2,
        grid=(B, L - 1, NCB),
        in_specs=[score_spec, x_spec, any_spec, any_spec, any_spec],
        out_specs=[x_spec, x_spec, x_spec],
        scratch_shapes=[pltpu.VMEM((8, HW), jnp.float32)] * 3,
    )

    outs = pl.pallas_call(
        functools.partial(_gate_body, L, HW, CB),
        grid_spec=grid_spec,
        out_shape=out_shapes,
        input_output_aliases={4: 0, 5: 1, 6: 2},
        compiler_params=pltpu.CompilerParams(
            dimension_semantics=("arbitrary", "arbitrary", "arbitrary")),
    )(keep, thr_flat, scores, x4, *outs0)

    return tuple(o.reshape(B, L, C, H, W) for o in outs)


# speedup vs baseline: 1.0046x; 1.0046x over previous
import functools

import jax
import jax.numpy as jnp
from jax import lax
from jax.experimental import pallas as pl
from jax.experimental.pallas import tpu as pltpu
from jax.experimental.pallas import tpu_sc as plsc

_THRESHOLD = 0.1
_ONE_BITS = 0x3F800001


def _sigmoid(z):
    return 1.0 / (1.0 + jnp.exp(-z))


def _score_body(score_ref, ego_ref, cav_ref, out_ref):
    ego = ego_ref[0, 0]
    cav = cav_ref[0, 0]
    del score_ref
    r = jnp.max(_sigmoid(cav - ego), axis=0, keepdims=True)
    a = jnp.max(_sigmoid(cav + ego), axis=0, keepdims=True)
    out_ref[0, 0] = lax.bitcast_convert_type(
        jnp.concatenate([r, a], axis=0), jnp.int32)


def _scores(psm4, B, L, P, HW):
    ego_spec = pl.BlockSpec((1, 1, P, HW), lambda b, ip: (b, 0, 0, 0))
    cav_spec = pl.BlockSpec((1, 1, P, HW), lambda b, ip: (b, ip + 1, 0, 0))
    out_spec = pl.BlockSpec((1, 1, 2, HW), lambda b, ip: (b, ip, 0, 0))
    return pl.pallas_call(
        functools.partial(_score_body, None),
        grid=(B, L - 1),
        in_specs=[ego_spec, cav_spec],
        out_specs=out_spec,
        out_shape=jax.ShapeDtypeStruct((B, L - 1, 2, HW), jnp.int32),
        compiler_params=pltpu.CompilerParams(
            dimension_semantics=("arbitrary", "arbitrary")),
    )(psm4, psm4)


def _sc_thresholds(scores2, NMAPS, HW, K):
    info = plsc.get_sparse_core_info()
    NC = info.num_cores
    mesh = plsc.VectorSubcoreMesh(core_axis_name="c", subcore_axis_name="s")
    n_chunk = HW // 16
    n_outer = n_chunk // 16

    def body(scores_hbm, thr_hbm, buf, tv):
        wid = lax.axis_index("s") * NC + lax.axis_index("c")

        @pl.when(wid < NMAPS)
        def _():
            pltpu.sync_copy(scores_hbm.at[wid], buf)
            one = jnp.ones((16,), jnp.int32)
            zero = jnp.zeros((16,), jnp.int32)

            del one

            def count_ge(mid):
                def inner(j, acc):
                    base = j * 256
                    for t in range(16):
                        v = buf[pl.ds(base + t * 16, 16)]
                        acc = acc + plsc.all_reduce_population_count(v >= mid)
                    return acc

                return lax.fori_loop(0, n_outer, inner, zero)

            def step(_, lohi):
                lo, hi = lohi
                mid = lo + lax.shift_right_logical(hi - lo, 1)
                pred = count_ge(mid) >= K
                return (jnp.where(pred, mid, lo), jnp.where(pred, hi, mid))

            lo, _ = lax.fori_loop(
                0, 31, step, (zero, jnp.full((16,), _ONE_BITS, jnp.int32)))
            tv[...] = lo
            pltpu.sync_copy(tv, thr_hbm.at[wid])

    run = pl.kernel(
        body,
        out_type=jax.ShapeDtypeStruct((NMAPS, 16), jnp.int32),
        mesh=mesh,
        scratch_types=[pltpu.VMEM((HW,), jnp.int32),
                       pltpu.VMEM((16,), jnp.int32)],
        compiler_params=pltpu.CompilerParams(needs_layout_passes=False),
    )
    return run(scores2)


def _gate_body(L, HW, CB, keep_ref, thr_ref, score_ref, x_ref, of_ref, or_ref,
               oa_ref, mm, mr, ma):
    b = pl.program_id(0)
    i = pl.program_id(1)
    cb = pl.program_id(2)

    @pl.when(cb == 0)
    def _build_masks():
        @pl.when(i == 0)
        def _ones():
            ones = jnp.ones((8, HW), jnp.float32)
            mm[:] = ones
            mr[:] = ones
            ma[:] = ones

        @pl.when(i != 0)
        def _from_thr():
            bits = score_ref[0, 0]
            t = (b * (L - 1) + (i - 1)) * 2
            kf = jnp.where(keep_ref[b * L + i] != 0, jnp.float32(1.0),
                           jnp.float32(0.0))
            fr = (bits[0:1] >= thr_ref[t]).astype(jnp.float32) * kf
            fa = (bits[1:2] >= thr_ref[t + 1]).astype(jnp.float32) * kf
            mr[:] = jnp.broadcast_to(fr, (8, HW))
            ma[:] = jnp.broadcast_to(fa, (8, HW))
            mm[:] = jnp.broadcast_to(jnp.maximum(fr, fa), (8, HW))

    mmv = mm[:]
    mrv = mr[:]
    mav = ma[:]
    for j in range(CB // 8):
        sl = pl.ds(j * 8, 8)
        xs = x_ref[0, 0, sl]
        of_ref[0, 0, sl] = xs * mmv
        or_ref[0, 0, sl] = xs * mrv
        oa_ref[0, 0, sl] = xs * mav


def kernel(x, psm, mask, flag):
    B, L, C, H, W = x.shape
    P = psm.shape[2]
    HW = H * W
    K = max(1, int(HW * _THRESHOLD))
    CB = 64 if C % 64 == 0 else C
    NCB = C // CB
    NMAPS = B * (L - 1) * 2

    x4 = x.reshape(B, L, C, HW)
    psm4 = psm.reshape(B, L, P, HW)
    keep = ((mask * jnp.asarray(flag, mask.dtype)) != 0).astype(
        jnp.int32).reshape(-1)

    scores = _scores(psm4, B, L, P, HW)
    thr = _sc_thresholds(scores.reshape(NMAPS, HW), NMAPS, HW, K)
    thr_flat = thr[:, 0]

    score_spec = pl.BlockSpec(
        (1, 1, 2, HW), lambda b, i, cb, *_: (b, jnp.maximum(i - 1, 0), 0, 0))
    x_spec = pl.BlockSpec((1, 1, CB, HW), lambda b, i, cb, *_: (b, i, cb, 0))

    grid_spec = pltpu.PrefetchScalarGridSpec(
        num_scalar_prefetch=2,
        grid=(B, L, NCB),
        in_specs=[score_spec, x_spec],
        out_specs=[x_spec, x_spec, x_spec],
        scratch_shapes=[pltpu.VMEM((8, HW), jnp.float32)] * 3,
    )

    outs = pl.pallas_call(
        functools.partial(_gate_body, L, HW, CB),
        grid_spec=grid_spec,
        out_shape=[jax.ShapeDtypeStruct((B, L, C, HW), jnp.float32)] * 3,
        compiler_params=pltpu.CompilerParams(
            dimension_semantics=("arbitrary", "arbitrary", "arbitrary")),
    )(keep, thr_flat, scores, x4)

    return tuple(o.reshape(B, L, C, H, W) for o in outs)
